# baseline (speedup 1.0000x reference)
import functools

import jax
import jax.numpy as jnp
from jax import lax
from jax.experimental import pallas as pl
from jax.experimental.pallas import tpu as pltpu
from jax.experimental.pallas import tpu_sc as plsc

_D = 32
_K = 1024
_B = 8
_T = 196
_N = _B * _T
_BPW = 56
_NWU = _N // _BPW


def _argmin_body(x_ref, w_ref, idx_ref):
    wm = w_ref[...]
    wn = jnp.sum(wm * wm, axis=0)[:, None]
    xc = jnp.concatenate(
        [x_ref[b] for b in range(_B)], axis=1
    )
    dots = lax.dot_general(
        wm, xc, (((0,), (0,)), ((), ())),
        preferred_element_type=jnp.float32,
        precision=lax.Precision.HIGHEST,
    )
    s = wn - 2.0 * dots
    idx_ref[...] = jnp.argmin(s, axis=0).astype(jnp.int32)[None, :]


_tc_argmin = pl.pallas_call(
    _argmin_body,
    out_shape=jax.ShapeDtypeStruct((1, _N), jnp.int32),
)


def _transpose_body(rows_ref, out_ref):
    rows = rows_ref[...]
    out_ref[...] = rows.reshape(_B, _T, _D).transpose(0, 2, 1)


_tc_transpose = pl.pallas_call(
    _transpose_body,
    out_shape=jax.ShapeDtypeStruct((_B, _D, _T), jnp.float32),
)


def _sc_gather_body(table_hbm, idx_hbm, out_hbm, idx_v, rows_v, sem):
    wid = lax.axis_index("s") * 2 + lax.axis_index("c")

    @pl.when(wid < _NWU)
    def _():
        base = pl.multiple_of(wid * _BPW, 8)
        pltpu.sync_copy(idx_hbm.at[pl.ds(base, _BPW)], idx_v)
        pltpu.async_copy(table_hbm.at[idx_v], rows_v, sem).wait()
        pltpu.sync_copy(rows_v, out_hbm.at[pl.ds(base, _BPW)])


@functools.cache
def _sc_gather():
    return functools.partial(
        pl.kernel,
        mesh=plsc.VectorSubcoreMesh(core_axis_name="c", subcore_axis_name="s"),
        out_type=jax.ShapeDtypeStruct((_N, _D), jnp.float32),
        scratch_types=[
            pltpu.VMEM((_BPW,), jnp.int32),
            pltpu.VMEM((_BPW, _D), jnp.float32),
            pltpu.SemaphoreType.DMA,
        ],
        compiler_params=pltpu.CompilerParams(use_tc_tiling_on_sc=False),
    )(_sc_gather_body)


def kernel(x, weight):
    B, d, h, w = x.shape
    table, x = lax.optimization_barrier((weight.T, x))
    idx = _tc_argmin(x.reshape(B, d, h * w), weight)
    rows = _sc_gather()(table, idx.reshape(_N))
    result = _tc_transpose(rows).reshape(B, d, h, w)
    return result, idx.reshape(B, 1, h, w)


# speedup vs baseline: 1.1749x; 1.1749x over previous
import functools

import jax
import jax.numpy as jnp
from jax import lax
from jax.experimental import pallas as pl
from jax.experimental.pallas import tpu as pltpu
from jax.experimental.pallas import tpu_sc as plsc

_D = 32
_K = 1024
_B = 8
_T = 196
_N = _B * _T
_BPW = 56
_NWU = _N // _BPW


def _argmin_body(x_ref, w_ref, idx_ref):
    wm = w_ref[...]
    wn = jnp.sum(wm * wm, axis=0)[:, None]
    xc = jnp.concatenate(
        [x_ref[b] for b in range(_B)], axis=1
    )
    dots = lax.dot_general(
        wm, xc, (((0,), (0,)), ((), ())),
        preferred_element_type=jnp.float32,
        precision=lax.Precision.HIGHEST,
    )
    s = wn - 2.0 * dots
    idx_ref[...] = jnp.argmin(s, axis=0).astype(jnp.int32)


_tc_argmin = pl.pallas_call(
    _argmin_body,
    out_shape=jax.ShapeDtypeStruct((_N,), jnp.int32),
)


def _sc_gather_body(table_hbm, idx_hbm, out_hbm, idx_v, rows_v, sem):
    wid = lax.axis_index("s") * 2 + lax.axis_index("c")

    @pl.when(wid < _NWU)
    def _():
        base = pl.multiple_of(wid * _BPW, 8)
        pltpu.sync_copy(idx_hbm.at[pl.ds(base, _BPW)], idx_v)
        pltpu.async_copy(table_hbm.at[idx_v], rows_v, sem).wait()
        pltpu.sync_copy(rows_v, out_hbm.at[pl.ds(base, _BPW)])


@functools.cache
def _sc_gather():
    return functools.partial(
        pl.kernel,
        mesh=plsc.VectorSubcoreMesh(core_axis_name="c", subcore_axis_name="s"),
        out_type=jax.ShapeDtypeStruct((_N, _D), jnp.float32),
        scratch_types=[
            pltpu.VMEM((_BPW,), jnp.int32),
            pltpu.VMEM((_BPW, _D), jnp.float32),
            pltpu.SemaphoreType.DMA,
        ],
        compiler_params=pltpu.CompilerParams(use_tc_tiling_on_sc=False),
    )(_sc_gather_body)


def kernel(x, weight):
    B, d, h, w = x.shape
    idx = _tc_argmin(x.reshape(B, d, h * w), weight)
    rows = _sc_gather()(weight.T, idx)
    result = rows.reshape(B, h, w, d).transpose(0, 3, 1, 2)
    return result, idx.reshape(B, 1, h, w)
